# baseline (speedup 1.0000x reference)
import functools

import jax
import jax.numpy as jnp
from jax import lax
from jax.experimental import pallas as pl
from jax.experimental.pallas import tpu as pltpu
from jax.experimental.pallas import tpu_sc as plsc

D_MODEL = 128
N_TABLE = 2048
NC, NS = 2, 16
NW = NC * NS
CHUNK = 128
J = 2
SUPER = J * CHUNK
NBUF = 2


def _make_gather(B):
    b_per_w = B // NW
    n_super = b_per_w // SUPER
    assert n_super % NBUF == 0 and n_super > NBUF
    n_groups = n_super // NBUF
    mesh = plsc.VectorSubcoreMesh(core_axis_name="c", subcore_axis_name="s")

    @functools.partial(
        pl.kernel,
        out_type=jax.ShapeDtypeStruct((B, D_MODEL), jnp.float32),
        mesh=mesh,
        scratch_types=[
            pltpu.VMEM((b_per_w,), jnp.int32),
            pltpu.VMEM_SHARED((N_TABLE, D_MODEL), jnp.float32),
            *[pltpu.VMEM((SUPER, D_MODEL), jnp.float32) for _ in range(NBUF)],
            *[pltpu.SemaphoreType.DMA for _ in range(2 * NBUF)],
        ],
    )
    def gather_kernel(idx_hbm, pe_hbm, out_hbm, idx_v, table_sh, *bufs_and_sems):
        rows = bufs_and_sems[:NBUF]
        gsem = bufs_and_sems[NBUF:2 * NBUF]
        wsem = bufs_and_sems[2 * NBUF:]
        sid = lax.axis_index("s")
        wid = sid * NC + lax.axis_index("c")
        base = wid * b_per_w

        t_rows = N_TABLE // NS
        pltpu.sync_copy(pe_hbm.at[pl.ds(sid * t_rows, t_rows)],
                        table_sh.at[pl.ds(sid * t_rows, t_rows)])
        pltpu.sync_copy(idx_hbm.at[pl.ds(base, b_per_w)], idx_v)
        plsc.subcore_barrier()

        def start_gathers(b, si):
            for j in range(J):
                off = pl.multiple_of(si * SUPER + j * CHUNK, CHUNK)
                pltpu.make_async_copy(
                    table_sh.at[idx_v.at[pl.ds(off, CHUNK)]],
                    rows[b].at[pl.ds(j * CHUNK, CHUNK)], gsem[b]).start()

        def wait_gathers(b):
            pltpu.make_async_copy(
                pe_hbm.at[pl.ds(0, SUPER)], rows[b], gsem[b]).wait()

        def write_desc(b, si):
            off = pl.multiple_of(si * SUPER, SUPER)
            return pltpu.make_async_copy(
                rows[b], out_hbm.at[pl.ds(base + off, SUPER)], wsem[b])

        start_gathers(0, 0)

        def group(g, carry):
            for b in range(NBUF):
                i = g * NBUF + b
                b2 = (b + 1) % NBUF
                wait_gathers(b)
                write_desc(b, i).start()

                @pl.when(i >= 1)
                def _():
                    write_desc(b2, i - 1).wait()

                nxt = i + 1

                @pl.when(nxt < n_super)
                def _():
                    start_gathers(b2, nxt)
            return carry

        lax.fori_loop(0, n_groups, group, 0)
        write_desc((n_super - 1) % NBUF, n_super - 1).wait()

    return gather_kernel


_B_TOTAL = 4096 * 200
_gather = _make_gather(_B_TOTAL)


def kernel(t, pe):
    idx = t.reshape(-1).astype(jnp.int32)
    out = _gather(idx, pe)
    return out.reshape(t.shape + (D_MODEL,))


# speedup vs baseline: 1.0339x; 1.0339x over previous
import functools

import jax
import jax.numpy as jnp
from jax import lax
from jax.experimental import pallas as pl
from jax.experimental.pallas import tpu as pltpu
from jax.experimental.pallas import tpu_sc as plsc

D_MODEL = 128
N_TABLE = 2048
NC, NS = 2, 16
NW = NC * NS
CHUNK = 128
NBUF = 5
LAG = 1


def _make_gather(B):
    b_per_w = B // NW
    n_chunks = b_per_w // CHUNK
    assert n_chunks % NBUF == 0 and n_chunks > NBUF
    n_groups = n_chunks // NBUF
    mesh = plsc.VectorSubcoreMesh(core_axis_name="c", subcore_axis_name="s")

    @functools.partial(
        pl.kernel,
        out_type=jax.ShapeDtypeStruct((B, D_MODEL), jnp.float32),
        mesh=mesh,
        scratch_types=[
            pltpu.VMEM((b_per_w,), jnp.int32),
            pltpu.VMEM_SHARED((N_TABLE, D_MODEL), jnp.float32),
            *[pltpu.VMEM((CHUNK, D_MODEL), jnp.float32) for _ in range(NBUF)],
            *[pltpu.SemaphoreType.DMA for _ in range(2 * NBUF)],
        ],
    )
    def gather_kernel(idx_hbm, pe_hbm, out_hbm, idx_v, table_sh, *bufs_and_sems):
        rows = bufs_and_sems[:NBUF]
        gsem = bufs_and_sems[NBUF:2 * NBUF]
        wsem = bufs_and_sems[2 * NBUF:]
        sid = lax.axis_index("s")
        wid = sid * NC + lax.axis_index("c")
        base = wid * b_per_w

        t_rows = N_TABLE // NS
        pltpu.sync_copy(pe_hbm.at[pl.ds(sid * t_rows, t_rows)],
                        table_sh.at[pl.ds(sid * t_rows, t_rows)])
        pltpu.sync_copy(idx_hbm.at[pl.ds(base, b_per_w)], idx_v)
        plsc.subcore_barrier()

        def gather_desc(b, ci):
            off = pl.multiple_of(ci * CHUNK, CHUNK)
            return pltpu.make_async_copy(
                table_sh.at[idx_v.at[pl.ds(off, CHUNK)]], rows[b], gsem[b])

        def write_desc(b, ci):
            off = pl.multiple_of(ci * CHUNK, CHUNK)
            return pltpu.make_async_copy(
                rows[b], out_hbm.at[pl.ds(base + off, CHUNK)], wsem[b])

        for b in range(NBUF - LAG):
            gather_desc(b, b).start()

        def group(g, carry):
            for b in range(NBUF):
                i = g * NBUF + b
                b2 = (b + NBUF - LAG) % NBUF
                gather_desc(b, i).wait()
                write_desc(b, i).start()

                @pl.when(i >= LAG)
                def _():
                    write_desc(b2, i - LAG).wait()

                nxt = i + NBUF - LAG

                @pl.when(nxt < n_chunks)
                def _():
                    gather_desc(b2, nxt).start()
            return carry

        lax.fori_loop(0, n_groups, group, 0)

        for j in range(LAG):
            ci = n_chunks - LAG + j
            write_desc(ci % NBUF, ci).wait()

    return gather_kernel


_B_TOTAL = 4096 * 200
_gather = _make_gather(_B_TOTAL)


def kernel(t, pe):
    idx = t.reshape(-1).astype(jnp.int32)
    out = _gather(idx, pe)
    return out.reshape(t.shape + (D_MODEL,))


# speedup vs baseline: 1.0371x; 1.0031x over previous
import functools

import jax
import jax.numpy as jnp
from jax import lax
from jax.experimental import pallas as pl
from jax.experimental.pallas import tpu as pltpu
from jax.experimental.pallas import tpu_sc as plsc

D_MODEL = 128
N_TABLE = 2048
NC, NS = 2, 16
NW = NC * NS
CHUNK = 80
NBUF = 8
LAG = 3


def _make_gather(B):
    b_per_w = B // NW
    n_chunks = b_per_w // CHUNK
    assert n_chunks % NBUF == 0 and n_chunks > NBUF
    n_groups = n_chunks // NBUF
    mesh = plsc.VectorSubcoreMesh(core_axis_name="c", subcore_axis_name="s")

    @functools.partial(
        pl.kernel,
        out_type=jax.ShapeDtypeStruct((B, D_MODEL), jnp.float32),
        mesh=mesh,
        scratch_types=[
            pltpu.VMEM((b_per_w,), jnp.int32),
            pltpu.VMEM_SHARED((N_TABLE, D_MODEL), jnp.float32),
            *[pltpu.VMEM((CHUNK, D_MODEL), jnp.float32) for _ in range(NBUF)],
            *[pltpu.SemaphoreType.DMA for _ in range(2 * NBUF)],
        ],
    )
    def gather_kernel(idx_hbm, pe_hbm, out_hbm, idx_v, table_sh, *bufs_and_sems):
        rows = bufs_and_sems[:NBUF]
        gsem = bufs_and_sems[NBUF:2 * NBUF]
        wsem = bufs_and_sems[2 * NBUF:]
        sid = lax.axis_index("s")
        wid = sid * NC + lax.axis_index("c")
        base = wid * b_per_w

        t_rows = N_TABLE // NS
        pltpu.sync_copy(pe_hbm.at[pl.ds(sid * t_rows, t_rows)],
                        table_sh.at[pl.ds(sid * t_rows, t_rows)])
        pltpu.sync_copy(idx_hbm.at[pl.ds(base, b_per_w)], idx_v)
        plsc.subcore_barrier()

        def gather_desc(b, ci):
            off = pl.multiple_of(ci * CHUNK, CHUNK)
            return pltpu.make_async_copy(
                table_sh.at[idx_v.at[pl.ds(off, CHUNK)]], rows[b], gsem[b])

        def write_desc(b, ci):
            off = pl.multiple_of(ci * CHUNK, CHUNK)
            return pltpu.make_async_copy(
                rows[b], out_hbm.at[pl.ds(base + off, CHUNK)], wsem[b])

        for b in range(NBUF - LAG):
            gather_desc(b, b).start()

        def group(g, carry):
            for b in range(NBUF):
                i = g * NBUF + b
                b2 = (b + NBUF - LAG) % NBUF
                gather_desc(b, i).wait()
                write_desc(b, i).start()

                @pl.when(i >= LAG)
                def _():
                    write_desc(b2, i - LAG).wait()

                nxt = i + NBUF - LAG

                @pl.when(nxt < n_chunks)
                def _():
                    gather_desc(b2, nxt).start()
            return carry

        lax.fori_loop(0, n_groups, group, 0)

        for j in range(LAG):
            ci = n_chunks - LAG + j
            write_desc(ci % NBUF, ci).wait()

    return gather_kernel


_B_TOTAL = 4096 * 200
_gather = _make_gather(_B_TOTAL)


def kernel(t, pe):
    idx = t.reshape(-1).astype(jnp.int32)
    out = _gather(idx, pe)
    return out.reshape(t.shape + (D_MODEL,))
